# Initial kernel scaffold; baseline (speedup 1.0000x reference)
import jax
import jax.numpy as jnp
from jax.experimental import pallas as pl


def kernel(revision, test, file_table, test_table, W, b):
    raise NotImplementedError("write your pallas kernel here")


# speedup vs baseline: 9.2386x; 9.2386x over previous
import functools

import jax
import jax.numpy as jnp
from jax import lax
from jax.experimental import pallas as pl
from jax.experimental.pallas import tpu as pltpu
from jax.experimental.pallas import tpu_sc as plsc

B = 4096
LEN = 200
EMB = 50
NC = 2
NS = 16
NW = NC * NS
RW = B // NW
NBUF = 4
HALF = LEN // 2


def _rsqrt(x):
    i = plsc.bitcast(x, jnp.int32)
    i = jnp.int32(0x5F3759DF) - (i >> 1)
    y = plsc.bitcast(i, jnp.float32)
    for _ in range(3):
        y = y * (1.5 - 0.5 * x * y * y)
    return y


def _sc_body(ft_hbm, tt_hbm, rev_hbm, tidx_hbm, wv_hbm, bv_hbm, out_hbm,
             idx_v, tidx_v, test_v, gbuf, dd_v, outb, wv_v, bv_v,
             sem_t, *gsems):
    wid = lax.axis_index("s") * NC + lax.axis_index("c")
    base = wid * RW

    pltpu.sync_copy(rev_hbm.at[pl.ds(base, RW)], idx_v)
    pltpu.sync_copy(tidx_hbm.at[wid], tidx_v)
    pltpu.sync_copy(wv_hbm, wv_v)
    pltpu.sync_copy(bv_hbm, bv_v)
    pltpu.async_copy(tt_hbm.at[tidx_v], test_v, sem_t).wait()

    def issue(i, b):
        pltpu.async_copy(ft_hbm.at[idx_v.at[i, 0]],
                         gbuf.at[b, pl.ds(0, HALF)], gsems[b])
        pltpu.async_copy(ft_hbm.at[idx_v.at[i, 1]],
                         gbuf.at[b, pl.ds(HALF, HALF)], gsems[b])

    def drain(i, b):
        pltpu.make_async_copy(ft_hbm.at[idx_v.at[i, 0]],
                              gbuf.at[b, pl.ds(0, HALF)], gsems[b]).wait()
        pltpu.make_async_copy(ft_hbm.at[idx_v.at[i, 1]],
                              gbuf.at[b, pl.ds(HALF, HALF)], gsems[b]).wait()

    lane = lax.iota(jnp.int32, 16)
    tailmask = lane >= 14

    def compute(i, b):
        zero = jnp.zeros((16,), jnp.float32)

        def acc_body(r, acc):
            a0, a1, a2, a3 = acc
            a0 = a0 + gbuf[b, r, pl.ds(0, 16)]
            a1 = a1 + gbuf[b, r, pl.ds(16, 16)]
            a2 = a2 + gbuf[b, r, pl.ds(32, 16)]
            a3 = a3 + gbuf[b, r, pl.ds(34, 16)]
            return a0, a1, a2, a3

        a0, a1, a2, a3 = lax.fori_loop(0, LEN, acc_body,
                                       (zero, zero, zero, zero), unroll=8)
        t0 = test_v[i, pl.ds(0, 16)]
        t1 = test_v[i, pl.ds(16, 16)]
        t2 = test_v[i, pl.ds(32, 16)]
        t3 = test_v[i, pl.ds(34, 16)]
        a3 = jnp.where(tailmask, a3, 0.0)
        t3 = jnp.where(tailmask, t3, 0.0)
        d0 = jnp.sum(a0 * t0 + a1 * t1 + a2 * t2 + a3 * t3)
        d1 = jnp.sum(a0 * a0 + a1 * a1 + a2 * a2 + a3 * a3)
        d2 = jnp.sum(t0 * t0 + t1 * t1 + t2 * t2 + t3 * t3)
        v = jnp.where(lane == 0, d0,
                      jnp.where(lane == 1, d1,
                                jnp.where(lane == 2, d2, 0.0)))
        dd_v[i, :] = v

    for b in range(NBUF):
        issue(b, b)

    def step(s, carry):
        for b in range(NBUF):
            i = NBUF * s + b
            drain(i, b)
            compute(i, b)

            @pl.when(i + NBUF < RW)
            def _():
                issue(i + NBUF, b)
        return carry

    lax.fori_loop(0, RW // NBUF, step, 0)

    wv = wv_v[...]
    bv = bv_v[...]
    for c in range(RW // 16):
        ridx = 16 * c + lane
        e0 = plsc.load_gather(dd_v, [ridx, lane * 0])
        e1 = plsc.load_gather(dd_v, [ridx, lane * 0 + 1])
        e2 = plsc.load_gather(dd_v, [ridx, lane * 0 + 2])
        merged = e0 * _rsqrt(jnp.maximum(e1 * e2, 1e-30))
        z = merged * wv + bv
        outb[pl.ds(16 * c, 16)] = 1.0 / (1.0 + jnp.exp(-z))
    pltpu.sync_copy(outb, out_hbm.at[pl.ds(base, RW)])


@jax.jit
def kernel(revision, test, file_table, test_table, W, b):
    rev3 = revision.astype(jnp.int32).reshape(B, 2, HALF)
    tidx = test.astype(jnp.int32).reshape(NW, RW)
    wv = jnp.full((16,), W[0, 0], jnp.float32)
    bv = jnp.full((16,), b[0], jnp.float32)

    call = pl.kernel(
        _sc_body,
        out_type=jax.ShapeDtypeStruct((B,), jnp.float32),
        mesh=plsc.VectorSubcoreMesh(core_axis_name="c", subcore_axis_name="s"),
        compiler_params=pltpu.CompilerParams(needs_layout_passes=False,
                                             use_tc_tiling_on_sc=False),
        scratch_types=[
            pltpu.VMEM((RW, 2, HALF), jnp.int32),
            pltpu.VMEM((RW,), jnp.int32),
            pltpu.VMEM((RW, EMB), jnp.float32),
            pltpu.VMEM((NBUF, LEN, EMB), jnp.float32),
            pltpu.VMEM((RW, 16), jnp.float32),
            pltpu.VMEM((RW,), jnp.float32),
            pltpu.VMEM((16,), jnp.float32),
            pltpu.VMEM((16,), jnp.float32),
            pltpu.SemaphoreType.DMA,
        ] + [pltpu.SemaphoreType.DMA] * NBUF,
    )
    out = call(file_table, test_table, rev3, tidx, wv, bv)
    return out.reshape(B, 1)


# speedup vs baseline: 10.0266x; 1.0853x over previous
import jax
import jax.numpy as jnp
from jax import lax
from jax.experimental import pallas as pl
from jax.experimental.pallas import tpu as pltpu
from jax.experimental.pallas import tpu_sc as plsc

B = 4096
LEN = 200
EMB = 50
EMBP = 64
NC = 2
NS = 16
NW = NC * NS
RW = B // NW
NBUF = 4
S0, S1 = 104, 96


def _rsqrt(x):
    i = plsc.bitcast(x, jnp.int32)
    i = jnp.int32(0x5F3759DF) - (i >> 1)
    y = plsc.bitcast(i, jnp.float32)
    for _ in range(3):
        y = y * (1.5 - 0.5 * x * y * y)
    return y


def _sc_body(ft_hbm, tt_hbm, rev_hbm, test_hbm, w_hbm, b_hbm, out_hbm,
             idxa_v, idxb_v, tidx_v, test_v, gbuf, dd_v, outb, w_s, b_s,
             sem_t, *gsems):
    wid = lax.axis_index("s") * NC + lax.axis_index("c")
    base = wid * RW

    pltpu.sync_copy(rev_hbm.at[pl.ds(base, RW), pl.ds(0, S0)], idxa_v)
    pltpu.sync_copy(rev_hbm.at[pl.ds(base, RW), pl.ds(S0, S1)], idxb_v)
    pltpu.sync_copy(test_hbm.at[pl.ds(base, RW)], tidx_v)
    pltpu.sync_copy(w_hbm, w_s)
    pltpu.sync_copy(b_hbm, b_s)
    pltpu.async_copy(tt_hbm.at[tidx_v], test_v, sem_t).wait()

    def issue(i, b):
        pltpu.async_copy(ft_hbm.at[idxa_v.at[i]],
                         gbuf.at[b, pl.ds(0, S0)], gsems[b])
        pltpu.async_copy(ft_hbm.at[idxb_v.at[i]],
                         gbuf.at[b, pl.ds(S0, S1)], gsems[b])

    def drain(i, b):
        pltpu.make_async_copy(ft_hbm.at[idxa_v.at[i]],
                              gbuf.at[b, pl.ds(0, S0)], gsems[b]).wait()
        pltpu.make_async_copy(ft_hbm.at[idxb_v.at[i]],
                              gbuf.at[b, pl.ds(S0, S1)], gsems[b]).wait()

    lane = lax.iota(jnp.int32, 16)

    def compute(i, b):
        zero = jnp.zeros((16,), jnp.float32)

        def acc_body(r, acc):
            a0, a1, a2, a3 = acc
            a0 = a0 + gbuf[b, r, pl.ds(0, 16)]
            a1 = a1 + gbuf[b, r, pl.ds(16, 16)]
            a2 = a2 + gbuf[b, r, pl.ds(32, 16)]
            a3 = a3 + gbuf[b, r, pl.ds(48, 16)]
            return a0, a1, a2, a3

        a0, a1, a2, a3 = lax.fori_loop(0, LEN, acc_body,
                                       (zero, zero, zero, zero), unroll=8)
        t0 = test_v[i, pl.ds(0, 16)]
        t1 = test_v[i, pl.ds(16, 16)]
        t2 = test_v[i, pl.ds(32, 16)]
        t3 = test_v[i, pl.ds(48, 16)]
        d0 = jnp.sum(a0 * t0 + a1 * t1 + a2 * t2 + a3 * t3)
        d1 = jnp.sum(a0 * a0 + a1 * a1 + a2 * a2 + a3 * a3)
        d2 = jnp.sum(t0 * t0 + t1 * t1 + t2 * t2 + t3 * t3)
        v = jnp.where(lane == 0, d0,
                      jnp.where(lane == 1, d1,
                                jnp.where(lane == 2, d2, 0.0)))
        dd_v[i, :] = v

    for b in range(NBUF):
        issue(b, b)

    def step(s, carry):
        for b in range(NBUF):
            i = NBUF * s + b
            drain(i, b)
            compute(i, b)

            @pl.when(i + NBUF < RW)
            def _():
                issue(i + NBUF, b)
        return carry

    lax.fori_loop(0, RW // NBUF, step, 0)

    wv = w_s[...]
    bv = b_s[...]
    for c in range(RW // 16):
        ridx = 16 * c + lane
        e0 = plsc.load_gather(dd_v, [ridx, lane * 0])
        e1 = plsc.load_gather(dd_v, [ridx, lane * 0 + 1])
        e2 = plsc.load_gather(dd_v, [ridx, lane * 0 + 2])
        merged = e0 * _rsqrt(jnp.maximum(e1 * e2, 1e-30))
        z = merged * wv + bv
        outb[pl.ds(16 * c, 16)] = 1.0 / (1.0 + jnp.exp(-z))
    pltpu.sync_copy(outb, out_hbm.at[pl.ds(base, RW)])


@jax.jit
def kernel(revision, test, file_table, test_table, W, b):
    call = pl.kernel(
        _sc_body,
        out_type=jax.ShapeDtypeStruct((B,), jnp.float32),
        mesh=plsc.VectorSubcoreMesh(core_axis_name="c", subcore_axis_name="s"),
        compiler_params=pltpu.CompilerParams(needs_layout_passes=False,
                                             use_tc_tiling_on_sc=False),
        scratch_types=[
            pltpu.VMEM((RW, S0), jnp.int32),
            pltpu.VMEM((RW, S1), jnp.int32),
            pltpu.VMEM((RW,), jnp.int32),
            pltpu.VMEM((RW, EMBP), jnp.float32),
            pltpu.VMEM((NBUF, LEN, EMBP), jnp.float32),
            pltpu.VMEM((RW, 16), jnp.float32),
            pltpu.VMEM((RW,), jnp.float32),
            pltpu.VMEM((16,), jnp.float32),
            pltpu.VMEM((16,), jnp.float32),
            pltpu.SemaphoreType.DMA,
        ] + [pltpu.SemaphoreType.DMA] * NBUF,
    )
    wv = jnp.full((16,), W[0, 0], jnp.float32)
    bv = jnp.full((16,), b[0], jnp.float32)
    ftp = jnp.pad(file_table, ((0, 0), (0, EMBP - EMB)))
    ttp = jnp.pad(test_table, ((0, 0), (0, EMBP - EMB)))
    out = call(ftp, ttp, revision.astype(jnp.int32),
               test.astype(jnp.int32).reshape(B), wv, bv)
    return out.reshape(B, 1)


# speedup vs baseline: 10.1384x; 1.0111x over previous
import jax
import jax.numpy as jnp
from jax import lax
from jax.experimental import pallas as pl
from jax.experimental.pallas import tpu as pltpu
from jax.experimental.pallas import tpu_sc as plsc

B = 4096
LEN = 200
EMB = 50
EMBP = 64
NUM_ROWS = 100000
NC = 2
NS = 16
NW = NC * NS
RW = B // NW
NBUF = 4
S0, S1 = 104, 96


def _rsqrt(x):
    i = plsc.bitcast(x, jnp.int32)
    i = jnp.int32(0x5F3759DF) - (i >> 1)
    y = plsc.bitcast(i, jnp.float32)
    for _ in range(3):
        y = y * (1.5 - 0.5 * x * y * y)
    return y


CH = 184
NCH = 17
LASTC = (100000 + CH - 1) // CH - 1
NLAST = 100000 - LASTC * CH


def _pad_body(tt1_hbm, ttp_hbm, a1d, ap):
    wid = lax.axis_index("s") * NC + lax.axis_index("c")

    def restride(n):
        def row(r, carry):
            ap[r, pl.ds(0, 16)] = a1d[pl.ds(50 * r, 16)]
            ap[r, pl.ds(16, 16)] = a1d[pl.ds(50 * r + 16, 16)]
            ap[r, pl.ds(32, 16)] = a1d[pl.ds(50 * r + 32, 16)]
            ap[r, pl.ds(48, 16)] = a1d[pl.ds(50 * r + 48, 16)]
            return carry
        lax.fori_loop(0, n, row, 0, unroll=4)

    for src_hbm, dst_hbm in ((tt1_hbm, ttp_hbm),):
        def chunk(j, carry):
            c = wid * NCH + j
            r0 = c * CH

            @pl.when(c < LASTC)
            def _():
                pltpu.sync_copy(src_hbm.at[pl.ds(r0 * 50, CH * 50)],
                                a1d.at[pl.ds(0, CH * 50)])
                restride(CH)
                pltpu.sync_copy(ap, dst_hbm.at[pl.ds(r0, CH)])

            @pl.when(c == LASTC)
            def _():
                pltpu.sync_copy(src_hbm.at[pl.ds(r0 * 50, NLAST * 50)],
                                a1d.at[pl.ds(0, NLAST * 50)])
                restride(NLAST)
                pltpu.sync_copy(ap.at[pl.ds(0, NLAST)],
                                dst_hbm.at[pl.ds(r0, NLAST)])
            return carry
        lax.fori_loop(0, NCH, chunk, 0)


def _sc_body(ft_hbm, tt_hbm, rev_hbm, test_hbm, w_hbm, b_hbm, out_hbm,
             idxa_v, idxb_v, tidx_v, test_v, gbuf, dd_v, outb, w_s, b_s,
             sem_t, *gsems):
    wid = lax.axis_index("s") * NC + lax.axis_index("c")
    base = wid * RW

    pltpu.sync_copy(rev_hbm.at[pl.ds(base, RW), pl.ds(0, S0)], idxa_v)
    pltpu.sync_copy(rev_hbm.at[pl.ds(base, RW), pl.ds(S0, S1)], idxb_v)
    pltpu.sync_copy(test_hbm.at[pl.ds(base, RW)], tidx_v)
    pltpu.sync_copy(w_hbm, w_s)
    pltpu.sync_copy(b_hbm, b_s)
    pltpu.async_copy(tt_hbm.at[tidx_v], test_v, sem_t).wait()

    def issue(i, b):
        pltpu.async_copy(ft_hbm.at[idxa_v.at[i]],
                         gbuf.at[b, pl.ds(0, S0)], gsems[b])
        pltpu.async_copy(ft_hbm.at[idxb_v.at[i]],
                         gbuf.at[b, pl.ds(S0, S1)], gsems[b])

    def drain(i, b):
        pltpu.make_async_copy(ft_hbm.at[idxa_v.at[i]],
                              gbuf.at[b, pl.ds(0, S0)], gsems[b]).wait()
        pltpu.make_async_copy(ft_hbm.at[idxb_v.at[i]],
                              gbuf.at[b, pl.ds(S0, S1)], gsems[b]).wait()

    lane = lax.iota(jnp.int32, 16)
    padmask = lane < 2

    def compute(i, b):
        zero = jnp.zeros((16,), jnp.float32)

        def acc_body(r, acc):
            a0, a1, a2, a3 = acc
            a0 = a0 + gbuf[b, r, pl.ds(0, 16)]
            a1 = a1 + gbuf[b, r, pl.ds(16, 16)]
            a2 = a2 + gbuf[b, r, pl.ds(32, 16)]
            a3 = a3 + gbuf[b, r, pl.ds(48, 16)]
            return a0, a1, a2, a3

        a0, a1, a2, a3 = lax.fori_loop(0, LEN, acc_body,
                                       (zero, zero, zero, zero), unroll=8)
        t0 = test_v[i, pl.ds(0, 16)]
        t1 = test_v[i, pl.ds(16, 16)]
        t2 = test_v[i, pl.ds(32, 16)]
        t3 = test_v[i, pl.ds(48, 16)]
        a3 = jnp.where(padmask, a3, 0.0)
        t3 = jnp.where(padmask, t3, 0.0)
        d0 = jnp.sum(a0 * t0 + a1 * t1 + a2 * t2 + a3 * t3)
        d1 = jnp.sum(a0 * a0 + a1 * a1 + a2 * a2 + a3 * a3)
        d2 = jnp.sum(t0 * t0 + t1 * t1 + t2 * t2 + t3 * t3)
        v = jnp.where(lane == 0, d0,
                      jnp.where(lane == 1, d1,
                                jnp.where(lane == 2, d2, 0.0)))
        dd_v[i, :] = v

    for b in range(NBUF):
        issue(b, b)

    def step(s, carry):
        for b in range(NBUF):
            i = NBUF * s + b
            drain(i, b)
            compute(i, b)

            @pl.when(i + NBUF < RW)
            def _():
                issue(i + NBUF, b)
        return carry

    lax.fori_loop(0, RW // NBUF, step, 0)

    wv = w_s[...]
    bv = b_s[...]
    for c in range(RW // 16):
        ridx = 16 * c + lane
        e0 = plsc.load_gather(dd_v, [ridx, lane * 0])
        e1 = plsc.load_gather(dd_v, [ridx, lane * 0 + 1])
        e2 = plsc.load_gather(dd_v, [ridx, lane * 0 + 2])
        merged = e0 * _rsqrt(jnp.maximum(e1 * e2, 1e-30))
        z = merged * wv + bv
        outb[pl.ds(16 * c, 16)] = 1.0 / (1.0 + jnp.exp(-z))
    pltpu.sync_copy(outb, out_hbm.at[pl.ds(base, RW)])


@jax.jit
def kernel(revision, test, file_table, test_table, W, b):
    pad_call = pl.kernel(
        _pad_body,
        out_type=jax.ShapeDtypeStruct((NUM_ROWS, EMBP), jnp.float32),
        mesh=plsc.VectorSubcoreMesh(core_axis_name="c", subcore_axis_name="s"),
        compiler_params=pltpu.CompilerParams(needs_layout_passes=False,
                                             use_tc_tiling_on_sc=False),
        scratch_types=[
            pltpu.VMEM((CH * 50 + 16,), jnp.float32),
            pltpu.VMEM((CH, EMBP), jnp.float32),
        ],
    )
    call = pl.kernel(
        _sc_body,
        out_type=jax.ShapeDtypeStruct((B,), jnp.float32),
        mesh=plsc.VectorSubcoreMesh(core_axis_name="c", subcore_axis_name="s"),
        compiler_params=pltpu.CompilerParams(needs_layout_passes=False,
                                             use_tc_tiling_on_sc=False),
        scratch_types=[
            pltpu.VMEM((RW, S0), jnp.int32),
            pltpu.VMEM((RW, S1), jnp.int32),
            pltpu.VMEM((RW,), jnp.int32),
            pltpu.VMEM((RW, EMBP), jnp.float32),
            pltpu.VMEM((NBUF, LEN, EMBP), jnp.float32),
            pltpu.VMEM((RW, 16), jnp.float32),
            pltpu.VMEM((RW,), jnp.float32),
            pltpu.VMEM((16,), jnp.float32),
            pltpu.VMEM((16,), jnp.float32),
            pltpu.SemaphoreType.DMA,
        ] + [pltpu.SemaphoreType.DMA] * NBUF,
    )
    wv = jnp.full((16,), W[0, 0], jnp.float32)
    bv = jnp.full((16,), b[0], jnp.float32)
    ttp = pad_call(test_table.reshape(-1))
    ftp = jnp.pad(file_table, ((0, 0), (0, EMBP - EMB)))
    out = call(ftp, ttp, revision.astype(jnp.int32),
               test.astype(jnp.int32).reshape(B), wv, bv)
    return out.reshape(B, 1)


# speedup vs baseline: 10.1536x; 1.0015x over previous
import jax
import jax.numpy as jnp
from jax import lax
from jax.experimental import pallas as pl
from jax.experimental.pallas import tpu as pltpu
from jax.experimental.pallas import tpu_sc as plsc

B = 4096
LEN = 200
EMB = 50
EMBP = 64
NUM_ROWS = 100000
NC = 2
NS = 16
NW = NC * NS
RW = B // NW
NBUF = 4
S0, S1 = 104, 96


def _rsqrt(x):
    i = plsc.bitcast(x, jnp.int32)
    i = jnp.int32(0x5F3759DF) - (i >> 1)
    y = plsc.bitcast(i, jnp.float32)
    for _ in range(3):
        y = y * (1.5 - 0.5 * x * y * y)
    return y


CH = 184
NCH = 17
LASTC = (100000 + CH - 1) // CH - 1
NLAST = 100000 - LASTC * CH


def _pad_body(tt1_hbm, ttp_hbm, a1d, ap):
    wid = lax.axis_index("s") * NC + lax.axis_index("c")

    def restride(n):
        def row(r, carry):
            ap[r, pl.ds(0, 16)] = a1d[pl.ds(50 * r, 16)]
            ap[r, pl.ds(16, 16)] = a1d[pl.ds(50 * r + 16, 16)]
            ap[r, pl.ds(32, 16)] = a1d[pl.ds(50 * r + 32, 16)]
            ap[r, pl.ds(48, 16)] = a1d[pl.ds(50 * r + 48, 16)]
            return carry
        lax.fori_loop(0, n, row, 0, unroll=4)

    for src_hbm, dst_hbm in ((tt1_hbm, ttp_hbm),):
        def chunk(j, carry):
            c = wid * NCH + j
            r0 = c * CH

            @pl.when(c < LASTC)
            def _():
                pltpu.sync_copy(src_hbm.at[pl.ds(r0 * 50, CH * 50)],
                                a1d.at[pl.ds(0, CH * 50)])
                restride(CH)
                pltpu.sync_copy(ap, dst_hbm.at[pl.ds(r0, CH)])

            @pl.when(c == LASTC)
            def _():
                pltpu.sync_copy(src_hbm.at[pl.ds(r0 * 50, NLAST * 50)],
                                a1d.at[pl.ds(0, NLAST * 50)])
                restride(NLAST)
                pltpu.sync_copy(ap.at[pl.ds(0, NLAST)],
                                dst_hbm.at[pl.ds(r0, NLAST)])
            return carry
        lax.fori_loop(0, NCH, chunk, 0)


def _sc_body(ft_hbm, tt_hbm, rev_hbm, test_hbm, w_hbm, b_hbm, out_hbm,
             idx_v, tidx_v, test_v, gbuf, dd_v, outb, w_s, b_s,
             sem_t, *gsems):
    wid = lax.axis_index("s") * NC + lax.axis_index("c")
    base = wid * RW

    pltpu.sync_copy(rev_hbm.at[pl.ds(base * LEN, RW * LEN)], idx_v)
    pltpu.sync_copy(test_hbm.at[pl.ds(base, RW)], tidx_v)
    pltpu.sync_copy(w_hbm, w_s)
    pltpu.sync_copy(b_hbm, b_s)
    pltpu.async_copy(tt_hbm.at[tidx_v], test_v, sem_t).wait()

    def issue(i, b):
        pltpu.async_copy(ft_hbm.at[idx_v.at[pl.ds(LEN * i, S0)]],
                         gbuf.at[b, pl.ds(0, S0)], gsems[b])
        pltpu.async_copy(ft_hbm.at[idx_v.at[pl.ds(LEN * i + S0, S1)]],
                         gbuf.at[b, pl.ds(S0, S1)], gsems[b])

    def drain(i, b):
        pltpu.make_async_copy(ft_hbm.at[idx_v.at[pl.ds(LEN * i, S0)]],
                              gbuf.at[b, pl.ds(0, S0)], gsems[b]).wait()
        pltpu.make_async_copy(ft_hbm.at[idx_v.at[pl.ds(LEN * i + S0, S1)]],
                              gbuf.at[b, pl.ds(S0, S1)], gsems[b]).wait()

    lane = lax.iota(jnp.int32, 16)
    padmask = lane < 2

    def compute(i, b):
        zero = jnp.zeros((16,), jnp.float32)

        def acc_body(r, acc):
            a0, a1, a2, a3 = acc
            a0 = a0 + gbuf[b, r, pl.ds(0, 16)]
            a1 = a1 + gbuf[b, r, pl.ds(16, 16)]
            a2 = a2 + gbuf[b, r, pl.ds(32, 16)]
            a3 = a3 + gbuf[b, r, pl.ds(48, 16)]
            return a0, a1, a2, a3

        a0, a1, a2, a3 = lax.fori_loop(0, LEN, acc_body,
                                       (zero, zero, zero, zero), unroll=8)
        t0 = test_v[i, pl.ds(0, 16)]
        t1 = test_v[i, pl.ds(16, 16)]
        t2 = test_v[i, pl.ds(32, 16)]
        t3 = test_v[i, pl.ds(48, 16)]
        a3 = jnp.where(padmask, a3, 0.0)
        t3 = jnp.where(padmask, t3, 0.0)
        d0 = jnp.sum(a0 * t0 + a1 * t1 + a2 * t2 + a3 * t3)
        d1 = jnp.sum(a0 * a0 + a1 * a1 + a2 * a2 + a3 * a3)
        d2 = jnp.sum(t0 * t0 + t1 * t1 + t2 * t2 + t3 * t3)
        v = jnp.where(lane == 0, d0,
                      jnp.where(lane == 1, d1,
                                jnp.where(lane == 2, d2, 0.0)))
        dd_v[i, :] = v

    for b in range(NBUF):
        issue(b, b)

    def step(s, carry):
        for b in range(NBUF):
            i = NBUF * s + b
            drain(i, b)
            compute(i, b)

            @pl.when(i + NBUF < RW)
            def _():
                issue(i + NBUF, b)
        return carry

    lax.fori_loop(0, RW // NBUF, step, 0)

    wv = w_s[...]
    bv = b_s[...]
    for c in range(RW // 16):
        ridx = 16 * c + lane
        e0 = plsc.load_gather(dd_v, [ridx, lane * 0])
        e1 = plsc.load_gather(dd_v, [ridx, lane * 0 + 1])
        e2 = plsc.load_gather(dd_v, [ridx, lane * 0 + 2])
        merged = e0 * _rsqrt(jnp.maximum(e1 * e2, 1e-30))
        z = merged * wv + bv
        outb[pl.ds(16 * c, 16)] = 1.0 / (1.0 + jnp.exp(-z))
    pltpu.sync_copy(outb, out_hbm.at[pl.ds(base, RW)])


@jax.jit
def kernel(revision, test, file_table, test_table, W, b):
    pad_call = pl.kernel(
        _pad_body,
        out_type=jax.ShapeDtypeStruct((NUM_ROWS, EMBP), jnp.float32),
        mesh=plsc.VectorSubcoreMesh(core_axis_name="c", subcore_axis_name="s"),
        compiler_params=pltpu.CompilerParams(needs_layout_passes=False,
                                             use_tc_tiling_on_sc=False),
        scratch_types=[
            pltpu.VMEM((CH * 50 + 16,), jnp.float32),
            pltpu.VMEM((CH, EMBP), jnp.float32),
        ],
    )
    call = pl.kernel(
        _sc_body,
        out_type=jax.ShapeDtypeStruct((B,), jnp.float32),
        mesh=plsc.VectorSubcoreMesh(core_axis_name="c", subcore_axis_name="s"),
        compiler_params=pltpu.CompilerParams(needs_layout_passes=False,
                                             use_tc_tiling_on_sc=False),
        scratch_types=[
            pltpu.VMEM((RW * LEN,), jnp.int32),
            pltpu.VMEM((RW,), jnp.int32),
            pltpu.VMEM((RW, EMBP), jnp.float32),
            pltpu.VMEM((NBUF, LEN, EMBP), jnp.float32),
            pltpu.VMEM((RW, 16), jnp.float32),
            pltpu.VMEM((RW,), jnp.float32),
            pltpu.VMEM((16,), jnp.float32),
            pltpu.VMEM((16,), jnp.float32),
            pltpu.SemaphoreType.DMA,
        ] + [pltpu.SemaphoreType.DMA] * NBUF,
    )
    wv = jnp.full((16,), W[0, 0], jnp.float32)
    bv = jnp.full((16,), b[0], jnp.float32)
    ttp = pad_call(test_table.reshape(-1))
    ftp = jnp.pad(file_table, ((0, 0), (0, EMBP - EMB)))
    out = call(ftp, ttp, revision.astype(jnp.int32).reshape(B * LEN),
               test.astype(jnp.int32).reshape(B), wv, bv)
    return out.reshape(B, 1)
